# Initial kernel scaffold; baseline (speedup 1.0000x reference)
import jax
import jax.numpy as jnp
from jax.experimental import pallas as pl


def kernel(cls_score, label, weight):
    raise NotImplementedError("write your pallas kernel here")


# speedup vs baseline: 218.9625x; 218.9625x over previous
import jax
import jax.numpy as jnp
from jax import lax
from jax.experimental import pallas as pl
from jax.experimental.pallas import tpu as pltpu
from jax.experimental.pallas import tpu_sc as plsc

N = 65536
C = 80
BG = 80
RATIO = 3

NS = 16
LAB_CH = N // NS
GR = 128
CH_RS = GR * C
NB = 32768
SL = NB // NS
LCAP = 2048
BM = 4096

_INT_MIN_PY = -2**31


def _dense_body(x_ref, lab_ref, w_ref, rs_ref, scal_ref):
    i = pl.program_id(0)
    xb = x_ref[...]
    lab = lab_ref[...]
    w = w_ref[...]
    pos = (lab >= 0) & (lab < BG)
    neg = lab == BG
    sp = jnp.maximum(xb, 0.0) + jnp.log1p(jnp.exp(-jnp.abs(xb)))
    s = jnp.sum(sp, axis=1, keepdims=True)
    oh = lax.broadcasted_iota(jnp.int32, (BM, C), 1) == lab
    xg = jnp.sum(jnp.where(oh, xb, 0.0), axis=1, keepdims=True)
    rs = w * (s - jnp.where(pos, xg, 0.0))
    rs_ref[...] = rs
    np_b = jnp.sum(pos.astype(jnp.float32))
    nn_b = jnp.sum(neg.astype(jnp.float32))
    pc_b = jnp.sum(jnp.where(pos, rs, 0.0))
    li = lax.broadcasted_iota(jnp.int32, (1, 128), 1)
    part = jnp.where(li == 0, np_b,
                     jnp.where(li == 1, nn_b,
                               jnp.where(li == 2, pc_b, 0.0)))

    @pl.when(i == 0)
    def _():
        scal_ref[...] = part

    @pl.when(i > 0)
    def _():
        scal_ref[...] = scal_ref[...] + part


def _dense(x, lab2, w2):
    grid = (N // BM,)
    return pl.pallas_call(
        _dense_body,
        grid=grid,
        in_specs=[
            pl.BlockSpec((BM, C), lambda i: (i, 0)),
            pl.BlockSpec((BM, 1), lambda i: (i, 0)),
            pl.BlockSpec((BM, 1), lambda i: (i, 0)),
        ],
        out_specs=[
            pl.BlockSpec((BM, 1), lambda i: (i, 0)),
            pl.BlockSpec((1, 128), lambda i: (0, 0)),
        ],
        out_shape=[
            jax.ShapeDtypeStruct((N, 1), jnp.float32),
            jax.ShapeDtypeStruct((1, 128), jnp.float32),
        ],
    )(x, lab2, w2)


def _binof(v):
    imin = jnp.int32(_INT_MIN_PY)
    bits = plsc.bitcast(v, jnp.int32)
    key = jnp.where(bits < 0, ~bits, bits | imin)
    key = key ^ imin
    return (key >> 17) + jnp.int32(NB // 2)


def _lanevec(vals, dtype):
    io = lax.iota(jnp.int32, 16)
    v = jnp.zeros((16,), dtype)
    for i, sc in enumerate(vals):
        v = jnp.where(io == i, jnp.full((16,), sc, dtype), v)
    return v


def _zero_ref(ref, nwords):
    z = jnp.zeros((16,), ref.dtype)

    def zb(i, _):
        ref[pl.ds(i * 16, 16)] = z
        return 0

    lax.fori_loop(0, nwords // 16, zb, 0)


def _select_body(x_hbm, lab_hbm, ids_hbm, info_hbm,
                 sh_hist, sh_misc, sh_misc2,
                 lab_v, ids_v, hist_v, idx_v, xbuf, comb_v, tmp_v,
                 m16_v, info_v, all_v, sem):
    t = lax.axis_index("s")
    base = t * LAB_CH
    pltpu.sync_copy(lab_hbm.at[pl.ds(base, LAB_CH)], lab_v)
    _zero_ref(ids_v, LAB_CH)
    _zero_ref(hist_v, NB)

    def cscan(i, carry):
        cnt, cpos = carry
        lab = lab_v[pl.ds(i * 16, 16)]
        negm = lab == BG
        posm = (lab >= 0) & (lab < BG)
        rowid = base + i * 16 + lax.iota(jnp.int32, 16)
        plsc.store_compressed(ids_v.at[pl.ds(cnt, 16)], rowid, mask=negm)
        cnt = cnt + plsc.all_reduce_population_count(negm)[0]
        cpos = cpos + plsc.all_reduce_population_count(posm)[0]
        return cnt, cpos

    cnt, cpos = lax.fori_loop(0, LAB_CH // 16, cscan,
                              (jnp.int32(0), jnp.int32(0)))

    ones_i = jnp.full((16,), 1, jnp.int32)
    nch = (cnt + GR - 1) // GR

    def hchunk(g, _):
        def cpidx(vi, _):
            idx_v[pl.ds(vi * 16, 16)] = ids_v[pl.ds(g * GR + vi * 16, 16)]
            return 0

        lax.fori_loop(0, GR // 16, cpidx, 0)
        pltpu.async_copy(x_hbm.at[idx_v], xbuf, sem).wait()
        nrow = jnp.minimum(GR, cnt - g * GR)

        def hrow(r, _):
            for c5 in range(5):
                v = xbuf[r, pl.ds(c5 * 16, 16)]
                plsc.addupdate_scatter(hist_v, [_binof(v)], ones_i)
            return 0

        lax.fori_loop(0, nrow, hrow, 0)
        return 0

    lax.fori_loop(0, nch, hchunk, 0)

    pltpu.sync_copy(hist_v, sh_hist.at[t])
    m16_v[...] = _lanevec([cnt, cpos], jnp.int32)
    pltpu.sync_copy(m16_v, sh_misc.at[pl.ds(t * 16, 16)])
    plsc.subcore_barrier()

    _zero_ref(comb_v, SL)
    for tp in range(NS):
        pltpu.sync_copy(sh_hist.at[tp, pl.ds(t * SL, SL)], tmp_v)

        def addc(i, _):
            comb_v[pl.ds(i * 16, 16)] = (comb_v[pl.ds(i * 16, 16)]
                                         + tmp_v[pl.ds(i * 16, 16)])
            return 0

        lax.fori_loop(0, SL // 16, addc, 0)

    pltpu.sync_copy(sh_misc, all_v)
    accv = jnp.zeros((16,), jnp.int32)
    for tp in range(NS):
        accv = accv + all_v[pl.ds(tp * 16, 16)]
    m = accv[0]
    p = accv[1]
    k = jnp.minimum(RATIO * p, m)

    def sumc(i, a):
        return a + comb_v[pl.ds(i * 16, 16)]

    s_slice = jnp.sum(lax.fori_loop(0, SL // 16, sumc,
                                    jnp.zeros((16,), jnp.int32)))
    m16_v[...] = _lanevec([s_slice], jnp.int32)
    pltpu.sync_copy(m16_v, sh_misc2.at[pl.ds(t * 16, 16)])
    plsc.subcore_barrier()
    pltpu.sync_copy(sh_misc2, all_v)
    accv = jnp.zeros((16,), jnp.int32)
    for tp in range(NS):
        accv = accv + jnp.where(t < tp, all_v[pl.ds(tp * 16, 16)], 0)
    a_t = accv[0]

    def walk(stp, carry):
        run, bst, c1 = carry
        vi = SL // 16 - 1 - stp
        v = comb_v[pl.ds(vi * 16, 16)]
        pc = plsc.cumsum(v)
        stot = jnp.sum(v)
        above = run + stot - pc
        win = (above < k) & (k <= above + v)
        bing = t * SL + vi * 16 + lax.iota(jnp.int32, 16)
        bst = jnp.maximum(bst, jnp.max(jnp.where(win, bing, -1)))
        c1 = jnp.maximum(c1, jnp.max(jnp.where(win, above, -1)))
        return run + stot, bst, c1

    _, bst, c1 = lax.fori_loop(0, SL // 16, walk,
                               (a_t, jnp.int32(-1), jnp.int32(-1)))
    q = jnp.where(bst >= 0, k - jnp.maximum(c1, 0), 0)
    info_v[...] = _lanevec([cnt, cpos, bst, c1, q, m, p, k], jnp.int32)
    pltpu.sync_copy(info_v, info_hbm.at[pl.ds(t * 16, 16)])
    pltpu.sync_copy(ids_v, ids_hbm.at[t])


def _select(x, lab):
    mesh = plsc.VectorSubcoreMesh(core_axis_name="c", subcore_axis_name="s",
                                  num_cores=1, num_subcores=NS)
    fn = pl.kernel(
        _select_body,
        out_type=(jax.ShapeDtypeStruct((NS, LAB_CH), jnp.int32),
                  jax.ShapeDtypeStruct((NS * 16,), jnp.int32)),
        mesh=mesh,
        compiler_params=pltpu.CompilerParams(
            needs_layout_passes=False, use_tc_tiling_on_sc=False),
        scratch_types=[
            pltpu.VMEM_SHARED((NS, NB), jnp.int32),
            pltpu.VMEM_SHARED((NS * 16,), jnp.int32),
            pltpu.VMEM_SHARED((NS * 16,), jnp.int32),
            pltpu.VMEM((LAB_CH,), jnp.int32),
            pltpu.VMEM((LAB_CH,), jnp.int32),
            pltpu.VMEM((NB,), jnp.int32),
            pltpu.VMEM((GR,), jnp.int32),
            pltpu.VMEM((GR, C), jnp.float32),
            pltpu.VMEM((SL,), jnp.int32),
            pltpu.VMEM((SL,), jnp.int32),
            pltpu.VMEM((16,), jnp.int32),
            pltpu.VMEM((16,), jnp.int32),
            pltpu.VMEM((NS * 16,), jnp.int32),
            pltpu.SemaphoreType.DMA,
        ],
    )
    return fn(x, lab)


def _eval_body(x_hbm, ids_hbm, info_hbm, rs_hbm, ascal_hbm, out_hbm,
               sh_e, sh_part,
               ids_v, idx_v, xbuf, rs_v, list_v, all_v, pall_v,
               m16i_v, m16f_v, asc_v, sem):
    t = lax.axis_index("s")
    pltpu.sync_copy(info_hbm, all_v)
    sumv = jnp.zeros((16,), jnp.int32)
    prefv = jnp.zeros((16,), jnp.int32)
    maxv = jnp.full((16,), -1, jnp.int32)
    for tp in range(NS):
        row = all_v[pl.ds(tp * 16, 16)]
        sumv = sumv + row
        prefv = prefv + jnp.where(t > tp, row, 0)
        maxv = jnp.maximum(maxv, row)
    row_own = all_v[pl.ds(t * 16, 16)]
    row0 = all_v[pl.ds(0, 16)]
    cnt = row_own[0]
    pref = prefv[0]
    bst = maxv[2]
    q = maxv[4]
    p = row0[6]
    k = row0[7]

    pltpu.sync_copy(ids_hbm.at[t], ids_v)
    pltpu.sync_copy(rs_hbm.at[pl.ds(N - 16, 16)], m16f_v)
    rs_last = m16f_v[...][15]

    zf = jnp.zeros((16,), jnp.float32)
    nch = (cnt + GR - 1) // GR

    def chunk(g, carry):
        acc, ecnt = carry

        def cpidx(vi, _):
            idx_v[pl.ds(vi * 16, 16)] = ids_v[pl.ds(g * GR + vi * 16, 16)]
            return 0

        lax.fori_loop(0, GR // 16, cpidx, 0)
        pltpu.async_copy(x_hbm.at[idx_v], xbuf, sem).wait()
        off = (pref + g * GR) * C
        offc = jnp.minimum(off, N - CH_RS)
        d = off - offc
        pltpu.sync_copy(rs_hbm.at[pl.ds(offc, CH_RS)],
                        rs_v.at[pl.ds(0, CH_RS)])
        nrow = jnp.minimum(GR, cnt - g * GR)

        def row(r, rc):
            racc, recnt = rc
            for c5 in range(5):
                v = xbuf[r, pl.ds(c5 * 16, 16)]
                b = _binof(v)
                s = d + r * C + c5 * 16
                sc = jnp.minimum(s, CH_RS)
                rsl = rs_v[pl.ds(sc, 16)]
                jv = off + r * C + c5 * 16 + lax.iota(jnp.int32, 16)
                rsv = jnp.where(jv < N, rsl, jnp.full((16,), rs_last))
                racc = racc + jnp.where(b > bst, rsv, 0.0)
                eq = b == bst
                ecs = jnp.minimum(recnt, LCAP - 16)
                plsc.store_compressed(list_v.at[pl.ds(ecs, 16)], rsv, mask=eq)
                recnt = recnt + plsc.all_reduce_population_count(eq)[0]
            return racc, recnt

        return lax.fori_loop(0, nrow, row, (acc, ecnt))

    acc, ecnt = lax.fori_loop(0, nch, chunk, (zf, jnp.int32(0)))
    sum_gt = jnp.sum(acc)

    m16i_v[...] = _lanevec([ecnt], jnp.int32)
    pltpu.sync_copy(m16i_v, sh_e.at[pl.ds(t * 16, 16)])
    plsc.subcore_barrier()
    pltpu.sync_copy(sh_e, all_v)
    ebv = jnp.zeros((16,), jnp.int32)
    for tp in range(NS):
        ebv = ebv + jnp.where(t > tp, all_v[pl.ds(tp * 16, 16)], 0)
    e_before = ebv[0]
    qt = jnp.clip(q - e_before, 0, jnp.minimum(ecnt, LCAP))

    def lsum(i, a):
        lv = list_v[pl.ds(i * 16, 16)]
        idx = i * 16 + lax.iota(jnp.int32, 16)
        return a + jnp.where(idx < qt, lv, 0.0)

    sum_eq = jnp.sum(lax.fori_loop(0, LCAP // 16, lsum, zf))
    part = sum_gt + sum_eq
    m16f_v[...] = _lanevec([part], jnp.float32)
    pltpu.sync_copy(m16f_v, sh_part.at[pl.ds(t * 16, 16)])
    plsc.subcore_barrier()

    @pl.when(t == 0)
    def _():
        pltpu.sync_copy(sh_part, pall_v)
        pltpu.sync_copy(ascal_hbm, asc_v)
        tot = jnp.zeros((16,), jnp.float32)
        for tp in range(NS):
            tot = tot + pall_v[pl.ds(tp * 16, 16)]
        total_neg = tot[0]
        pos_c = asc_v[pl.ds(0, 16)][2]
        count = ((k + p) * C).astype(jnp.float32)
        m16f_v[...] = _lanevec([total_neg + pos_c, count], jnp.float32)
        pltpu.sync_copy(m16f_v, out_hbm)


def _evaluate(x, ids, info, rs, ascal):
    mesh = plsc.VectorSubcoreMesh(core_axis_name="c", subcore_axis_name="s",
                                  num_cores=1, num_subcores=NS)
    fn = pl.kernel(
        _eval_body,
        out_type=jax.ShapeDtypeStruct((16,), jnp.float32),
        mesh=mesh,
        compiler_params=pltpu.CompilerParams(
            needs_layout_passes=False, use_tc_tiling_on_sc=False),
        scratch_types=[
            pltpu.VMEM_SHARED((NS * 16,), jnp.int32),
            pltpu.VMEM_SHARED((NS * 16,), jnp.float32),
            pltpu.VMEM((LAB_CH,), jnp.int32),
            pltpu.VMEM((GR,), jnp.int32),
            pltpu.VMEM((GR, C), jnp.float32),
            pltpu.VMEM((CH_RS + 16,), jnp.float32),
            pltpu.VMEM((LCAP,), jnp.float32),
            pltpu.VMEM((NS * 16,), jnp.int32),
            pltpu.VMEM((NS * 16,), jnp.float32),
            pltpu.VMEM((16,), jnp.int32),
            pltpu.VMEM((16,), jnp.float32),
            pltpu.VMEM((128,), jnp.float32),
            pltpu.SemaphoreType.DMA,
        ],
    )
    return fn(x, ids, info, rs, ascal)


def kernel(cls_score, label, weight):
    lab2 = label.reshape(N, 1)
    w2 = weight.reshape(N, 1)
    rs2, scal = _dense(cls_score, lab2, w2)
    ids, info = _select(cls_score, label)
    out = _evaluate(cls_score, ids, info, rs2.reshape(N), scal.reshape(128))
    return out[0] / out[1]


# speedup vs baseline: 258.0503x; 1.1785x over previous
import jax
import jax.numpy as jnp
from jax import lax
from jax.experimental import pallas as pl
from jax.experimental.pallas import tpu as pltpu
from jax.experimental.pallas import tpu_sc as plsc

N = 65536
C = 80
BG = 80
RATIO = 3

NS = 16
LAB_CH = N // NS
GR = 128
CH_RS = GR * C
NB = 8192
SL = NB // NS
LCAP = 512
BM = 4096

_INT_MIN_PY = -2**31


def _dense_body(x_ref, lab_ref, w_ref, rs_ref, scal_ref):
    i = pl.program_id(0)
    xb = x_ref[...]
    lab = lab_ref[...]
    w = w_ref[...]
    pos = (lab >= 0) & (lab < BG)
    neg = lab == BG
    sp = jnp.maximum(xb, 0.0) + jnp.log1p(jnp.exp(-jnp.abs(xb)))
    ohp = (lax.broadcasted_iota(jnp.int32, (BM, C), 1) == lab) & pos
    rs = w * jnp.sum(jnp.where(ohp, sp - xb, sp), axis=1, keepdims=True)
    rs_ref[...] = rs
    np_b = jnp.sum(pos.astype(jnp.float32))
    nn_b = jnp.sum(neg.astype(jnp.float32))
    pc_b = jnp.sum(jnp.where(pos, rs, 0.0))
    li = lax.broadcasted_iota(jnp.int32, (1, 128), 1)
    part = jnp.where(li == 0, np_b,
                     jnp.where(li == 1, nn_b,
                               jnp.where(li == 2, pc_b, 0.0)))

    @pl.when(i == 0)
    def _():
        scal_ref[...] = part

    @pl.when(i > 0)
    def _():
        scal_ref[...] = scal_ref[...] + part


def _dense(x, lab2, w2):
    grid = (N // BM,)
    return pl.pallas_call(
        _dense_body,
        grid=grid,
        in_specs=[
            pl.BlockSpec((BM, C), lambda i: (i, 0)),
            pl.BlockSpec((BM, 1), lambda i: (i, 0)),
            pl.BlockSpec((BM, 1), lambda i: (i, 0)),
        ],
        out_specs=[
            pl.BlockSpec((BM, 1), lambda i: (i, 0)),
            pl.BlockSpec((1, 128), lambda i: (0, 0)),
        ],
        out_shape=[
            jax.ShapeDtypeStruct((N, 1), jnp.float32),
            jax.ShapeDtypeStruct((1, 128), jnp.float32),
        ],
    )(x, lab2, w2)


def _binof(v):
    imin = jnp.int32(_INT_MIN_PY)
    bits = plsc.bitcast(v, jnp.int32)
    key = jnp.where(bits < 0, ~bits, bits | imin)
    key = key ^ imin
    return (key >> 19) + jnp.int32(NB // 2)


def _lanevec(vals, dtype):
    io = lax.iota(jnp.int32, 16)
    v = jnp.zeros((16,), dtype)
    for i, sc in enumerate(vals):
        v = jnp.where(io == i, jnp.full((16,), sc, dtype), v)
    return v


def _zero_ref(ref, nwords):
    z = jnp.zeros((16,), ref.dtype)

    def zb(i, _):
        ref[pl.ds(i * 16, 16)] = z
        return 0

    lax.fori_loop(0, nwords // 16, zb, 0, unroll=8)


def _select_body(x_hbm, lab_hbm, info_hbm, bins_hbm,
                 sh_hist, sh_misc, sh_misc2,
                 lab_v, ids_v, hist_v, idx_v, xbuf, binb_v, comb_v, tmp_v,
                 m16_v, info_v, all_v, sem):
    t = lax.axis_index("s")
    base = t * LAB_CH
    pltpu.sync_copy(lab_hbm.at[pl.ds(base, LAB_CH)], lab_v)
    _zero_ref(ids_v, LAB_CH)
    _zero_ref(hist_v, NB)

    def cscan(i, carry):
        cnt, cpos = carry
        lab = lab_v[pl.ds(i * 16, 16)]
        negm = lab == BG
        posm = (lab >= 0) & (lab < BG)
        rowid = base + i * 16 + lax.iota(jnp.int32, 16)
        plsc.store_compressed(ids_v.at[pl.ds(cnt, 16)], rowid, mask=negm)
        cnt = cnt + plsc.all_reduce_population_count(negm)[0]
        cpos = cpos + plsc.all_reduce_population_count(posm)[0]
        return cnt, cpos

    cnt, cpos = lax.fori_loop(0, LAB_CH // 16, cscan,
                              (jnp.int32(0), jnp.int32(0)), unroll=4)

    ones_i = jnp.full((16,), 1, jnp.int32)
    nch = (cnt + GR - 1) // GR

    def hchunk(g, _):
        def cpidx(vi, _):
            idx_v[pl.ds(vi * 16, 16)] = ids_v[pl.ds(g * GR + vi * 16, 16)]
            return 0

        lax.fori_loop(0, GR // 16, cpidx, 0)
        pltpu.async_copy(x_hbm.at[idx_v], xbuf, sem).wait()
        nrow = jnp.minimum(GR, cnt - g * GR)

        def hrow(r, _):
            for c5 in range(5):
                v = xbuf[r, pl.ds(c5 * 16, 16)]
                b = _binof(v)
                binb_v[pl.ds(r * C + c5 * 16, 16)] = b
                plsc.addupdate_scatter(hist_v, [b], ones_i)
            return 0

        lax.fori_loop(0, nrow, hrow, 0)
        pltpu.sync_copy(binb_v, bins_hbm.at[t, pl.ds(g * CH_RS, CH_RS)])
        return 0

    lax.fori_loop(0, nch, hchunk, 0)

    pltpu.sync_copy(hist_v, sh_hist.at[t])
    m16_v[...] = _lanevec([cnt, cpos], jnp.int32)
    pltpu.sync_copy(m16_v, sh_misc.at[pl.ds(t * 16, 16)])
    plsc.subcore_barrier()

    _zero_ref(comb_v, SL)
    for tp in range(NS):
        pltpu.sync_copy(sh_hist.at[tp, pl.ds(t * SL, SL)], tmp_v)

        def addc(i, _):
            comb_v[pl.ds(i * 16, 16)] = (comb_v[pl.ds(i * 16, 16)]
                                         + tmp_v[pl.ds(i * 16, 16)])
            return 0

        lax.fori_loop(0, SL // 16, addc, 0, unroll=8)

    pltpu.sync_copy(sh_misc, all_v)
    accv = jnp.zeros((16,), jnp.int32)
    for tp in range(NS):
        accv = accv + all_v[pl.ds(tp * 16, 16)]
    m = accv[0]
    p = accv[1]
    k = jnp.minimum(RATIO * p, m)

    def sumc(i, a):
        return a + comb_v[pl.ds(i * 16, 16)]

    s_slice = jnp.sum(lax.fori_loop(0, SL // 16, sumc,
                                    jnp.zeros((16,), jnp.int32)))
    m16_v[...] = _lanevec([s_slice], jnp.int32)
    pltpu.sync_copy(m16_v, sh_misc2.at[pl.ds(t * 16, 16)])
    plsc.subcore_barrier()
    pltpu.sync_copy(sh_misc2, all_v)
    accv = jnp.zeros((16,), jnp.int32)
    for tp in range(NS):
        accv = accv + jnp.where(t < tp, all_v[pl.ds(tp * 16, 16)], 0)
    a_t = accv[0]

    def walk(stp, carry):
        run, bst, c1 = carry
        vi = SL // 16 - 1 - stp
        v = comb_v[pl.ds(vi * 16, 16)]
        pc = plsc.cumsum(v)
        stot = jnp.sum(v)
        above = run + stot - pc
        win = (above < k) & (k <= above + v)
        bing = t * SL + vi * 16 + lax.iota(jnp.int32, 16)
        bst = jnp.maximum(bst, jnp.max(jnp.where(win, bing, -1)))
        c1 = jnp.maximum(c1, jnp.max(jnp.where(win, above, -1)))
        return run + stot, bst, c1

    _, bst, c1 = lax.fori_loop(0, SL // 16, walk,
                               (a_t, jnp.int32(-1), jnp.int32(-1)))
    q = jnp.where(bst >= 0, k - jnp.maximum(c1, 0), 0)
    info_v[...] = _lanevec([cnt, cpos, bst, c1, q, m, p, k], jnp.int32)
    pltpu.sync_copy(info_v, info_hbm.at[pl.ds(t * 16, 16)])


def _select(x, lab):
    mesh = plsc.VectorSubcoreMesh(core_axis_name="c", subcore_axis_name="s",
                                  num_cores=1, num_subcores=NS)
    fn = pl.kernel(
        _select_body,
        out_type=(jax.ShapeDtypeStruct((NS * 16,), jnp.int32),
                  jax.ShapeDtypeStruct((NS, LAB_CH * C), jnp.int32)),
        mesh=mesh,
        compiler_params=pltpu.CompilerParams(
            needs_layout_passes=False, use_tc_tiling_on_sc=False),
        scratch_types=[
            pltpu.VMEM_SHARED((NS, NB), jnp.int32),
            pltpu.VMEM_SHARED((NS * 16,), jnp.int32),
            pltpu.VMEM_SHARED((NS * 16,), jnp.int32),
            pltpu.VMEM((LAB_CH,), jnp.int32),
            pltpu.VMEM((LAB_CH,), jnp.int32),
            pltpu.VMEM((NB,), jnp.int32),
            pltpu.VMEM((GR,), jnp.int32),
            pltpu.VMEM((GR, C), jnp.float32),
            pltpu.VMEM((CH_RS,), jnp.int32),
            pltpu.VMEM((SL,), jnp.int32),
            pltpu.VMEM((SL,), jnp.int32),
            pltpu.VMEM((16,), jnp.int32),
            pltpu.VMEM((16,), jnp.int32),
            pltpu.VMEM((NS * 16,), jnp.int32),
            pltpu.SemaphoreType.DMA,
        ],
    )
    return fn(x, lab)


def _eval_body(bins_hbm, info_hbm, rs_hbm, ascal_hbm, out_hbm,
               sh_e, sh_part,
               binb_v, rs_v, list_v, all_v, pall_v,
               m16i_v, m16f_v, asc_v):
    t = lax.axis_index("s")
    pltpu.sync_copy(info_hbm, all_v)
    sumv = jnp.zeros((16,), jnp.int32)
    prefv = jnp.zeros((16,), jnp.int32)
    maxv = jnp.full((16,), -1, jnp.int32)
    for tp in range(NS):
        row = all_v[pl.ds(tp * 16, 16)]
        sumv = sumv + row
        prefv = prefv + jnp.where(t > tp, row, 0)
        maxv = jnp.maximum(maxv, row)
    row_own = all_v[pl.ds(t * 16, 16)]
    row0 = all_v[pl.ds(0, 16)]
    cnt = row_own[0]
    pref = prefv[0]
    bst = maxv[2]
    q = maxv[4]
    p = row0[6]
    k = row0[7]

    pltpu.sync_copy(rs_hbm.at[pl.ds(N - 16, 16)], m16f_v)
    rs_last = m16f_v[...][15]

    zf = jnp.zeros((16,), jnp.float32)
    nch = (cnt + GR - 1) // GR

    def chunk(g, carry):
        acc, ecnt = carry
        pltpu.sync_copy(bins_hbm.at[t, pl.ds(g * CH_RS, CH_RS)], binb_v)
        off = (pref + g * GR) * C
        offc = jnp.minimum(off, N - CH_RS)
        d = off - offc
        pltpu.sync_copy(rs_hbm.at[pl.ds(offc, CH_RS)],
                        rs_v.at[pl.ds(0, CH_RS)])
        nrow = jnp.minimum(GR, cnt - g * GR)

        def row(r, rc):
            racc, recnt = rc
            for c5 in range(5):
                b = binb_v[pl.ds(r * C + c5 * 16, 16)]
                s = d + r * C + c5 * 16
                sc = jnp.minimum(s, CH_RS)
                rsl = rs_v[pl.ds(sc, 16)]
                jv = off + r * C + c5 * 16 + lax.iota(jnp.int32, 16)
                rsv = jnp.where(jv < N, rsl, jnp.full((16,), rs_last))
                racc = racc + jnp.where(b > bst, rsv, 0.0)
                eq = b == bst
                ecs = jnp.minimum(recnt, LCAP - 16)
                plsc.store_compressed(list_v.at[pl.ds(ecs, 16)], rsv, mask=eq)
                recnt = recnt + plsc.all_reduce_population_count(eq)[0]
            return racc, recnt

        return lax.fori_loop(0, nrow, row, (acc, ecnt))

    acc, ecnt = lax.fori_loop(0, nch, chunk, (zf, jnp.int32(0)))
    sum_gt = jnp.sum(acc)

    m16i_v[...] = _lanevec([ecnt], jnp.int32)
    pltpu.sync_copy(m16i_v, sh_e.at[pl.ds(t * 16, 16)])
    plsc.subcore_barrier()
    pltpu.sync_copy(sh_e, all_v)
    ebv = jnp.zeros((16,), jnp.int32)
    for tp in range(NS):
        ebv = ebv + jnp.where(t > tp, all_v[pl.ds(tp * 16, 16)], 0)
    e_before = ebv[0]
    qt = jnp.clip(q - e_before, 0, jnp.minimum(ecnt, LCAP))

    def lsum(i, a):
        lv = list_v[pl.ds(i * 16, 16)]
        idx = i * 16 + lax.iota(jnp.int32, 16)
        return a + jnp.where(idx < qt, lv, 0.0)

    sum_eq = jnp.sum(lax.fori_loop(0, LCAP // 16, lsum, zf, unroll=4))
    part = sum_gt + sum_eq
    m16f_v[...] = _lanevec([part], jnp.float32)
    pltpu.sync_copy(m16f_v, sh_part.at[pl.ds(t * 16, 16)])
    plsc.subcore_barrier()

    @pl.when(t == 0)
    def _():
        pltpu.sync_copy(sh_part, pall_v)
        pltpu.sync_copy(ascal_hbm, asc_v)
        tot = jnp.zeros((16,), jnp.float32)
        for tp in range(NS):
            tot = tot + pall_v[pl.ds(tp * 16, 16)]
        total_neg = tot[0]
        pos_c = asc_v[pl.ds(0, 16)][2]
        count = ((k + p) * C).astype(jnp.float32)
        m16f_v[...] = _lanevec([total_neg + pos_c, count], jnp.float32)
        pltpu.sync_copy(m16f_v, out_hbm)


def _evaluate(bins, info, rs, ascal):
    mesh = plsc.VectorSubcoreMesh(core_axis_name="c", subcore_axis_name="s",
                                  num_cores=1, num_subcores=NS)
    fn = pl.kernel(
        _eval_body,
        out_type=jax.ShapeDtypeStruct((16,), jnp.float32),
        mesh=mesh,
        compiler_params=pltpu.CompilerParams(
            needs_layout_passes=False, use_tc_tiling_on_sc=False),
        scratch_types=[
            pltpu.VMEM_SHARED((NS * 16,), jnp.int32),
            pltpu.VMEM_SHARED((NS * 16,), jnp.float32),
            pltpu.VMEM((CH_RS,), jnp.int32),
            pltpu.VMEM((CH_RS + 16,), jnp.float32),
            pltpu.VMEM((LCAP,), jnp.float32),
            pltpu.VMEM((NS * 16,), jnp.int32),
            pltpu.VMEM((NS * 16,), jnp.float32),
            pltpu.VMEM((16,), jnp.int32),
            pltpu.VMEM((16,), jnp.float32),
            pltpu.VMEM((128,), jnp.float32),
        ],
    )
    return fn(bins, info, rs, ascal)


def kernel(cls_score, label, weight):
    lab2 = label.reshape(N, 1)
    w2 = weight.reshape(N, 1)
    rs2, scal = _dense(cls_score, lab2, w2)
    info, bins = _select(cls_score, label)
    out = _evaluate(bins, info, rs2.reshape(N), scal.reshape(128))
    return out[0] / out[1]
